# baseline (speedup 1.0000x reference)
import functools
import math

import jax
import jax.numpy as jnp
from jax import lax
from jax.experimental import pallas as pl
from jax.experimental.pallas import tpu as pltpu
from jax.experimental.pallas import tpu_sc as plsc

D_MODEL = 64
SCALE = 1.0 / math.sqrt(D_MODEL)

C = 200


def kernel(x, lut):
    B1, B2 = x.shape
    V, D = lut.shape
    lutp = jnp.pad(lut, ((0, 0), (0, 2 * D - lut.shape[1])))
    flat_idx = x.reshape(B1 * B2).astype(jnp.int32)
    return _call(flat_idx, lutp, B1, B2, D)


@functools.partial(jax.jit, static_argnums=(2, 3, 4))
def _call(flat_idx, lutp, B1, B2, D):
    info = plsc.get_sparse_core_info()
    NC, NS = info.num_cores, info.num_subcores
    NW = NC * NS
    n_chunks = (B1 * B2) // (NW * C)
    mesh = plsc.VectorSubcoreMesh(core_axis_name="c", subcore_axis_name="s")

    def body(idx_hbm, table_hbm, out_hbm, idx_v, rows2_v, rows1_v, gsem, wsem):
        wid = lax.axis_index("s") * NC + lax.axis_index("c")
        base = wid * n_chunks

        def fetch_idx(b, ci):
            off = (base + ci) * C
            pltpu.sync_copy(idx_hbm.at[pl.ds(off, C)], idx_v[b])

        def fire_gather(b):
            pltpu.async_copy(table_hbm.at[idx_v[b]], rows2_v[b], gsem[b])

        def compact(b):
            def grp(i, carry):
                r0 = i * 8
                for u in range(8):
                    for j in range(D // 16):
                        src = rows2_v[b][r0 + u, pl.ds(j * 16, 16)]
                        rows1_v[b][r0 + u, pl.ds(j * 16, 16)] = src * SCALE
                return carry

            lax.fori_loop(0, C // 8, grp, 0)

        for b in range(2):
            fetch_idx(b, b)
            fire_gather(b)

        def super_body(s, carry):
            for b in range(2):
                ci = s * 2 + b
                pltpu.make_async_copy(table_hbm.at[idx_v[b]], rows2_v[b],
                                      gsem[b]).wait()
                fetch_idx(b, ci + 2)

                @pl.when(ci >= 2)
                def _():
                    pltpu.make_async_copy(
                        rows1_v[b], out_hbm.at[base + ci - 2], wsem[b]).wait()

                compact(b)
                pltpu.async_copy(rows1_v[b], out_hbm.at[base + ci], wsem[b])
                fire_gather(b)
            return carry

        lax.fori_loop(0, n_chunks // 2 - 1, super_body, 0)

        for b in range(2):
            ci = n_chunks - 2 + b
            pltpu.make_async_copy(table_hbm.at[idx_v[b]], rows2_v[b],
                                  gsem[b]).wait()
            pltpu.make_async_copy(rows1_v[b], out_hbm.at[base + ci - 2],
                                  wsem[b]).wait()
            compact(b)
            pltpu.async_copy(rows1_v[b], out_hbm.at[base + ci], wsem[b])
        for b in range(2):
            ci = n_chunks - 2 + b
            pltpu.make_async_copy(rows1_v[b], out_hbm.at[base + ci],
                                  wsem[b]).wait()

    return pl.kernel(
        body,
        mesh=mesh,
        compiler_params=pltpu.CompilerParams(use_tc_tiling_on_sc=True,
                                             needs_layout_passes=False),
        out_type=jax.ShapeDtypeStruct((B1, B2, D), jnp.float32),
        scratch_types=[
            [pltpu.VMEM((C,), jnp.int32) for _ in range(2)],
            [pltpu.VMEM((C, 2 * D), jnp.float32) for _ in range(2)],
            [pltpu.VMEM((C, D), jnp.float32) for _ in range(2)],
            [pltpu.SemaphoreType.DMA for _ in range(2)],
            [pltpu.SemaphoreType.DMA for _ in range(2)],
        ],
    )(flat_idx, lutp)


# speedup vs baseline: 1.1015x; 1.1015x over previous
import functools
import math

import jax
import jax.numpy as jnp
from jax import lax
from jax.experimental import pallas as pl
from jax.experimental.pallas import tpu as pltpu
from jax.experimental.pallas import tpu_sc as plsc

D_MODEL = 64
SCALE = 1.0 / math.sqrt(D_MODEL)

C = 200


def kernel(x, lut):
    B1, B2 = x.shape
    V, D = lut.shape
    lutp = jnp.pad(lut, ((0, 0), (0, 2 * D - lut.shape[1])))
    flat_idx = x.reshape(B1 * B2).astype(jnp.int32)
    return _call(flat_idx, lutp, B1, B2, D).reshape(B1, B2, D)


@functools.partial(jax.jit, static_argnums=(2, 3, 4))
def _call(flat_idx, lutp, B1, B2, D):
    info = plsc.get_sparse_core_info()
    NC, NS = info.num_cores, info.num_subcores
    NW = NC * NS
    n_chunks = (B1 * B2) // (NW * C)
    mesh = plsc.VectorSubcoreMesh(core_axis_name="c", subcore_axis_name="s")

    def body(idx_hbm, table_hbm, out_hbm, idx_v, rows2_v, rows1_v, gsem, wsem):
        wid = lax.axis_index("s") * NC + lax.axis_index("c")
        base = wid * n_chunks

        def fetch_idx(b, ci):
            off = (base + ci) * C
            pltpu.sync_copy(idx_hbm.at[pl.ds(off, C)], idx_v[b])

        def fire_gather(b):
            pltpu.async_copy(table_hbm.at[idx_v[b]], rows2_v[b], gsem[b])

        def compact(b):
            def grp(i, carry):
                r0 = i * 8
                for u in range(8):
                    for j in range(D // 16):
                        src = rows2_v[b][r0 + u, pl.ds(j * 16, 16)]
                        rows1_v[b][r0 + u, pl.ds(j * 16, 16)] = src * SCALE
                return carry

            lax.fori_loop(0, C // 8, grp, 0)

        for b in range(2):
            fetch_idx(b, b)
            fire_gather(b)

        def super_body(s, carry):
            for b in range(2):
                ci = s * 2 + b
                pltpu.make_async_copy(table_hbm.at[idx_v[b]], rows2_v[b],
                                      gsem[b]).wait()
                fetch_idx(b, ci + 2)

                @pl.when(ci >= 2)
                def _():
                    pltpu.make_async_copy(
                        rows1_v[b],
                        out_hbm.at[pl.ds((base + ci - 2) * C, C)],
                        wsem[b]).wait()

                compact(b)
                pltpu.async_copy(rows1_v[b],
                                 out_hbm.at[pl.ds((base + ci) * C, C)],
                                 wsem[b])
                fire_gather(b)
            return carry

        lax.fori_loop(0, n_chunks // 2 - 1, super_body, 0)

        for b in range(2):
            ci = n_chunks - 2 + b
            pltpu.make_async_copy(table_hbm.at[idx_v[b]], rows2_v[b],
                                  gsem[b]).wait()
            pltpu.make_async_copy(rows1_v[b],
                                  out_hbm.at[pl.ds((base + ci - 2) * C, C)],
                                  wsem[b]).wait()
            compact(b)
            pltpu.async_copy(rows1_v[b], out_hbm.at[pl.ds((base + ci) * C, C)],
                             wsem[b])
        for b in range(2):
            ci = n_chunks - 2 + b
            pltpu.make_async_copy(rows1_v[b],
                                  out_hbm.at[pl.ds((base + ci) * C, C)],
                                  wsem[b]).wait()

    return pl.kernel(
        body,
        mesh=mesh,
        compiler_params=pltpu.CompilerParams(use_tc_tiling_on_sc=True,
                                             needs_layout_passes=False),
        out_type=jax.ShapeDtypeStruct((B1 * B2, D), jnp.float32),
        scratch_types=[
            [pltpu.VMEM((C,), jnp.int32) for _ in range(2)],
            [pltpu.VMEM((C, 2 * D), jnp.float32) for _ in range(2)],
            [pltpu.VMEM((C, D), jnp.float32) for _ in range(2)],
            [pltpu.SemaphoreType.DMA for _ in range(2)],
            [pltpu.SemaphoreType.DMA for _ in range(2)],
        ],
    )(flat_idx, lutp)
